# baseline (speedup 1.0000x reference)
import functools

import jax
import jax.numpy as jnp
from jax import lax
from jax.experimental import pallas as pl
from jax.experimental.pallas import tpu as pltpu
from jax.experimental.pallas import tpu_sc as plsc

E = 800000
CIN = 11
H = 64
B = 128
NC = 2
NS = 16
NW = NC * NS
CPW = 200
CPW0 = 250
CPW1 = 150
EP = NW * CPW * B
NCH = EP // B
BT = 1024
SB = 10
NBUF = 3


@functools.lru_cache(maxsize=None)
def _make_gather(C):
    n_planes = 4 if C == 64 else 1
    out_shape = ((4, EP // 2, 128) if C == 64 else (EP // 2, 128))
    mesh = plsc.VectorSubcoreMesh(core_axis_name="c", subcore_axis_name="s",
                                  num_cores=NC)

    @functools.partial(
        pl.kernel,
        out_type=jax.ShapeDtypeStruct(out_shape, jnp.float32),
        mesh=mesh,
        compiler_params=pltpu.CompilerParams(use_tc_tiling_on_sc=False),
        scratch_types=[
            pltpu.VMEM((4, SB, B), jnp.int32),
            pltpu.VMEM((NBUF, 8, 64), jnp.int32),
            pltpu.VMEM((NBUF, 8, B // 2, C), jnp.float32),
        ] + [pltpu.SemaphoreType.DMA] * (2 * NBUF),
    )
    def gather_k(nb_hbm, h_hbm, out_hbm, raw_v, stage_v, gath_v, *sems):
        gsems, wsems = sems[:NBUF], sems[NBUF:]
        cidx = lax.axis_index("c")
        sidx = lax.axis_index("s")
        base = jnp.where(cidx == 0, sidx * CPW0, NS * CPW0 + sidx * CPW1)
        nsup = jnp.where(cidx == 0, CPW0 // SB, CPW1 // SB)

        def stage_idx(k, s):
            for par in range(2):
                for j in range(4):
                    for pb in range(4):
                        e = raw_v[j, k, pl.ds(par * 64 + pb * 16, 16)]
                        r = (((e >> 7) << 7) + 2 * (e & 63) + ((e >> 6) & 1))
                        stage_v[s, par * 4 + j, pl.ds(pb * 16, 16)] = r

        def super_body(t, carry):
            ch0 = base + t * SB
            for j in range(4):
                pltpu.sync_copy(nb_hbm.at[pl.ds(j * NCH + ch0, SB)],
                                raw_v.at[j])
            gcp, wcp = {}, {}

            def dst(kk, j, par):
                row0 = (ch0 + kk) * (B // 2)
                if C == 64:
                    return out_hbm.at[j, pl.ds(row0, B // 2),
                                      pl.ds(par * 64, 64)]
                return out_hbm.at[pl.ds(row0, B // 2),
                                  pl.ds(par * 64 + j * 16, 16)]

            def start_stores(kk):
                ss = kk % NBUF
                for c in gcp[kk]:
                    c.wait()
                wcp[kk] = [
                    pltpu.async_copy(gath_v.at[ss, par * 4 + j],
                                     dst(kk, j, par), wsems[ss])
                    for par in range(2) for j in range(4)
                ]

            for k in range(SB):
                s = k % NBUF
                if k >= NBUF:
                    for c in wcp[k - NBUF]:
                        c.wait()
                if C == 64:
                    stage_idx(k, s)
                    idx = lambda j, par: stage_v.at[s, par * 4 + j]
                else:
                    idx = lambda j, par: raw_v.at[j, k, pl.ds(par * 64, 64)]
                gcp[k] = [
                    pltpu.async_copy(h_hbm.at[idx(j, par)],
                                     gath_v.at[s, par * 4 + j], gsems[s])
                    for par in range(2) for j in range(4)
                ]
                if k >= 2:
                    start_stores(k - 2)
            start_stores(SB - 2)
            start_stores(SB - 1)
            for k in range(SB - NBUF, SB):
                for c in wcp[k]:
                    c.wait()
            return carry

        lax.fori_loop(0, nsup, super_body, None)

    return gather_k


def _ln_relu_pair(z, p_ref):
    halves = []
    for h0 in (0, 64):
        zz = z[:, h0:h0 + 64] + p_ref[0][None, :]
        m = jnp.mean(zz, axis=1, keepdims=True)
        zc = zz - m
        v = jnp.mean(zc * zc, axis=1, keepdims=True)
        halves.append(jnp.maximum(
            zc * lax.rsqrt(v + 1e-5) * p_ref[1][None, :]
            + p_ref[2][None, :], 0.0))
    return jnp.concatenate(halves, axis=1)


def _mid_act(h_ref, g_ref, w_ref, p_ref):
    hb = h_ref[...]
    g0, g1, g2, g3 = g_ref[0], g_ref[1], g_ref[2], g_ref[3]
    feats = (hb, g0 + g2, jnp.abs(g0 - g2), g1 + g3, jnp.abs(g1 - g3))
    z = None
    for k, f in enumerate(feats):
        zk = jnp.dot(f, w_ref[k * 128:(k + 1) * 128],
                     preferred_element_type=jnp.float32)
        z = zk if z is None else z + zk
    return _ln_relu_pair(z, p_ref) + hb


def _tc0_body(x_ref, g_ref, w_ref, p_ref, o_ref):
    xb = x_ref[...]
    gx = g_ref[...]
    pieces = [xb]
    for h0 in (0, 64):
        n0 = gx[:, h0:h0 + 16]
        n1 = gx[:, h0 + 16:h0 + 32]
        n2 = gx[:, h0 + 32:h0 + 48]
        n3 = gx[:, h0 + 48:h0 + 64]
        pieces += [n0 + n2, jnp.abs(n0 - n2), n1 + n3, jnp.abs(n1 - n3)]
    f = jnp.concatenate(pieces, axis=1)
    z = jnp.dot(f, w_ref[...], preferred_element_type=jnp.float32)
    o_ref[...] = _ln_relu_pair(z, p_ref)


def _tc_mid_body(h_ref, g_ref, w_ref, p_ref, o_ref):
    o_ref[...] = _mid_act(h_ref, g_ref, w_ref, p_ref)


def _tc_last_body(h_ref, g_ref, w_ref, p_ref, cw1_ref, w2_ref, hp_ref, o_ref):
    a = _mid_act(h_ref, g_ref, w_ref, p_ref)
    t = jnp.maximum(
        jnp.dot(a, cw1_ref[...], preferred_element_type=jnp.float32)
        + hp_ref[0][None, :], 0.0)
    o2 = jnp.dot(t, w2_ref[...], preferred_element_type=jnp.float32)
    o_ref[...] = o2[:, :2] + hp_ref[2, 0]


_R = BT // 2


def _mk_tc(body, g_spec, kw, out_shape, out_spec, extra_specs=()):
    return pl.pallas_call(
        body,
        grid=(EP // BT,),
        in_specs=[
            pl.BlockSpec((_R, 128), lambda i: (i, 0)),
            g_spec,
            pl.BlockSpec((kw, 128), lambda i: (0, 0)),
            pl.BlockSpec((8, H), lambda i: (0, 0)),
            *extra_specs,
        ],
        out_specs=out_spec,
        out_shape=out_shape,
    )


_TC0 = _mk_tc(_tc0_body,
              pl.BlockSpec((_R, 128), lambda i: (i, 0)), 256,
              jax.ShapeDtypeStruct((EP // 2, 128), jnp.float32),
              pl.BlockSpec((_R, 128), lambda i: (i, 0)))
_TCM = _mk_tc(_tc_mid_body,
              pl.BlockSpec((4, _R, 128), lambda i: (0, i, 0)), 640,
              jax.ShapeDtypeStruct((EP // 2, 128), jnp.float32),
              pl.BlockSpec((_R, 128), lambda i: (i, 0)))
_TCL = _mk_tc(_tc_last_body,
              pl.BlockSpec((4, _R, 128), lambda i: (0, i, 0)), 640,
              jax.ShapeDtypeStruct((EP // 2, 2), jnp.float32),
              pl.BlockSpec((_R, 2), lambda i: (i, 0)),
              extra_specs=(pl.BlockSpec((128, H), lambda i: (0, 0)),
                           pl.BlockSpec((H, 8), lambda i: (0, 0)),
                           pl.BlockSpec((8, H), lambda i: (0, 0))))


def _lr(Wk, side):
    z = jnp.zeros_like(Wk)
    return jnp.concatenate([Wk, z] if side == 0 else [z, Wk], axis=1)


def _wbig_mid(W):
    blocks = []
    for k in range(5):
        Wk = W[k * 64:(k + 1) * 64]
        blocks += [_lr(Wk, 0), _lr(Wk, 1)]
    return jnp.concatenate(blocks, axis=0)


def _wbig0(W0):
    wx = jnp.zeros((64, H), jnp.float32).at[:CIN].set(W0[:CIN])
    combo = [jnp.zeros((16, H), jnp.float32).at[:CIN].set(
        W0[k * CIN:(k + 1) * CIN]) for k in range(1, 5)]
    blocks = [_lr(wx, 0), _lr(wx, 1)]
    blocks += [_lr(c, 0) for c in combo]
    blocks += [_lr(c, 1) for c in combo]
    return jnp.concatenate(blocks, axis=0)


def kernel(x, neighbors, W0, b0, g0, be0, W1, b1, g1, be1,
           W2, b2, g2, be2, W3, b3, g3, be3, cW1, cb1, cW2, cb2):
    a = jnp.pad(x, ((0, EP - E), (0, 0))).reshape(NCH, 2, 64, CIN)
    z53 = jnp.zeros((NCH, 64, 53), jnp.float32)
    x64 = jnp.concatenate([a[:, 0], z53, a[:, 1], z53],
                          axis=-1).reshape(EP // 2, 128)
    xr8 = x.reshape(E // 8, 88)
    z5 = jnp.zeros((E // 8, 5), jnp.float32)
    p16 = []
    for s in range(8):
        p16 += [xr8[:, s * 11:(s + 1) * 11], z5]
    x16 = jnp.pad(jnp.concatenate(p16, axis=1),
                  ((0, (EP - E) // 8), (0, 0))).reshape(EP, 16)
    nbq = (jnp.pad(neighbors, ((0, EP - E), (0, 0))).T
           .reshape(4 * NCH, B))

    def pack(b, g, be):
        return jnp.concatenate(
            [b[None], g[None], be[None], jnp.zeros((5, H), jnp.float32)], 0)

    cw1b = jnp.zeros((128, 64), jnp.float32)
    cw1b = cw1b.at[:64, :32].set(cW1).at[64:, 32:].set(cW1)
    w2b = jnp.zeros((H, 8), jnp.float32)
    w2b = w2b.at[:32, 0].set(cW2[:, 0]).at[32:, 1].set(cW2[:, 0])
    hp = jnp.zeros((8, H), jnp.float32)
    hp = hp.at[0, :32].set(cb1).at[0, 32:].set(cb1)
    hp = hp.at[2, 0].set(cb2[0])

    g16, g64 = _make_gather(16), _make_gather(64)
    gx = g16(nbq, x16)
    h = _TC0(x64, gx, _wbig0(W0), pack(b0, g0, be0))
    for (W, b, g, be) in ((W1, b1, g1, be1), (W2, b2, g2, be2)):
        gp = g64(nbq, h.reshape(EP, 64))
        h = _TCM(h, gp, _wbig_mid(W), pack(b, g, be))
    gp = g64(nbq, h.reshape(EP, 64))
    out = _TCL(h, gp, _wbig_mid(W3), pack(b3, g3, be3), cw1b, w2b, hp)
    return out.reshape(NCH, 64, 2).transpose(0, 2, 1).reshape(EP)[:E]


# speedup vs baseline: 1.0207x; 1.0207x over previous
import functools

import jax
import jax.numpy as jnp
from jax import lax
from jax.experimental import pallas as pl
from jax.experimental.pallas import tpu as pltpu
from jax.experimental.pallas import tpu_sc as plsc

E = 800000
CIN = 11
H = 64
B = 128
NC = 2
NS = 16
NW = NC * NS
CPW = 200
NHALF = 2
CPWH = CPW // NHALF
EP = NW * CPW * B
NCH = EP // B
BT = 1024
SB = 10
NBUF = 3


@functools.lru_cache(maxsize=None)
def _make_gather(C, half):
    off = half * (NCH // NHALF)
    epl = EP // NHALF
    out_shape = ((4, epl // 2, 128) if C == 64 else (epl // 2, 128))
    mesh = plsc.VectorSubcoreMesh(core_axis_name="c", subcore_axis_name="s",
                                  num_cores=NC)

    @functools.partial(
        pl.kernel,
        out_type=jax.ShapeDtypeStruct(out_shape, jnp.float32),
        mesh=mesh,
        compiler_params=pltpu.CompilerParams(use_tc_tiling_on_sc=False),
        scratch_types=[
            pltpu.VMEM((4, SB, B), jnp.int32),
            pltpu.VMEM((NBUF, 8, 64), jnp.int32),
            pltpu.VMEM((NBUF, 8, B // 2, C), jnp.float32),
        ] + [pltpu.SemaphoreType.DMA] * (2 * NBUF),
    )
    def gather_k(nb_hbm, h_hbm, out_hbm, raw_v, stage_v, gath_v, *sems):
        gsems, wsems = sems[:NBUF], sems[NBUF:]
        wid = lax.axis_index("c") * NS + lax.axis_index("s")
        base = off + wid * CPWH

        def stage_idx(k, s):
            for par in range(2):
                for j in range(4):
                    for pb in range(4):
                        e = raw_v[j, k, pl.ds(par * 64 + pb * 16, 16)]
                        r = (((e >> 7) << 7) + 2 * (e & 63) + ((e >> 6) & 1))
                        stage_v[s, par * 4 + j, pl.ds(pb * 16, 16)] = r

        def super_body(t, carry):
            ch0 = base + t * SB
            for j in range(4):
                pltpu.sync_copy(nb_hbm.at[pl.ds(j * NCH + ch0, SB)],
                                raw_v.at[j])
            gcp, wcp = {}, {}

            def dst(kk, j, par):
                row0 = (ch0 - off + kk) * (B // 2)
                if C == 64:
                    return out_hbm.at[j, pl.ds(row0, B // 2),
                                      pl.ds(par * 64, 64)]
                return out_hbm.at[pl.ds(row0, B // 2),
                                  pl.ds(par * 64 + j * 16, 16)]

            def start_stores(kk):
                ss = kk % NBUF
                for c in gcp[kk]:
                    c.wait()
                wcp[kk] = [
                    pltpu.async_copy(gath_v.at[ss, par * 4 + j],
                                     dst(kk, j, par), wsems[ss])
                    for par in range(2) for j in range(4)
                ]

            for k in range(SB):
                s = k % NBUF
                if k >= NBUF:
                    for c in wcp[k - NBUF]:
                        c.wait()
                if C == 64:
                    stage_idx(k, s)
                    idx = lambda j, par: stage_v.at[s, par * 4 + j]
                else:
                    idx = lambda j, par: raw_v.at[j, k, pl.ds(par * 64, 64)]
                gcp[k] = [
                    pltpu.async_copy(h_hbm.at[idx(j, par)],
                                     gath_v.at[s, par * 4 + j], gsems[s])
                    for par in range(2) for j in range(4)
                ]
                if k >= 2:
                    start_stores(k - 2)
            start_stores(SB - 2)
            start_stores(SB - 1)
            for k in range(SB - NBUF, SB):
                for c in wcp[k]:
                    c.wait()
            return carry

        lax.fori_loop(0, CPWH // SB, super_body, None)

    return gather_k


def _ln_relu_pair(z, p_ref):
    halves = []
    for h0 in (0, 64):
        zz = z[:, h0:h0 + 64] + p_ref[0][None, :]
        m = jnp.mean(zz, axis=1, keepdims=True)
        zc = zz - m
        v = jnp.mean(zc * zc, axis=1, keepdims=True)
        halves.append(jnp.maximum(
            zc * lax.rsqrt(v + 1e-5) * p_ref[1][None, :]
            + p_ref[2][None, :], 0.0))
    return jnp.concatenate(halves, axis=1)


def _mid_act(h_ref, g_ref, w_ref, p_ref):
    hb = h_ref[...]
    g0, g1, g2, g3 = g_ref[0], g_ref[1], g_ref[2], g_ref[3]
    feats = (hb, g0 + g2, jnp.abs(g0 - g2), g1 + g3, jnp.abs(g1 - g3))
    z = None
    for k, f in enumerate(feats):
        zk = jnp.dot(f, w_ref[k * 128:(k + 1) * 128],
                     preferred_element_type=jnp.float32)
        z = zk if z is None else z + zk
    return _ln_relu_pair(z, p_ref) + hb


def _tc0_body(x_ref, g_ref, w_ref, p_ref, o_ref):
    xb = x_ref[...]
    gx = g_ref[...]
    pieces = [xb]
    for h0 in (0, 64):
        n0 = gx[:, h0:h0 + 16]
        n1 = gx[:, h0 + 16:h0 + 32]
        n2 = gx[:, h0 + 32:h0 + 48]
        n3 = gx[:, h0 + 48:h0 + 64]
        pieces += [n0 + n2, jnp.abs(n0 - n2), n1 + n3, jnp.abs(n1 - n3)]
    f = jnp.concatenate(pieces, axis=1)
    z = jnp.dot(f, w_ref[...], preferred_element_type=jnp.float32)
    o_ref[...] = _ln_relu_pair(z, p_ref)


def _tc_mid_body(h_ref, g_ref, w_ref, p_ref, o_ref):
    o_ref[...] = _mid_act(h_ref, g_ref, w_ref, p_ref)


def _tc_last_body(h_ref, g_ref, w_ref, p_ref, cw1_ref, w2_ref, hp_ref, o_ref):
    a = _mid_act(h_ref, g_ref, w_ref, p_ref)
    t = jnp.maximum(
        jnp.dot(a, cw1_ref[...], preferred_element_type=jnp.float32)
        + hp_ref[0][None, :], 0.0)
    o2 = jnp.dot(t, w2_ref[...], preferred_element_type=jnp.float32)
    o_ref[...] = o2[:, :2] + hp_ref[2, 0]


_R = BT // 2
_GH = EP // BT // NHALF


def _mk_tc(body, g_spec, kw, out_shape, out_spec, half, extra_specs=()):
    hoff = half * _GH
    return pl.pallas_call(
        body,
        grid=(_GH,),
        in_specs=[
            pl.BlockSpec((_R, 128), lambda i: (i + hoff, 0)),
            g_spec,
            pl.BlockSpec((kw, 128), lambda i: (0, 0)),
            pl.BlockSpec((8, H), lambda i: (0, 0)),
            *extra_specs,
        ],
        out_specs=out_spec,
        out_shape=out_shape,
    )


_EH = EP // NHALF
_TC0 = [_mk_tc(_tc0_body,
               pl.BlockSpec((_R, 128), lambda i: (i, 0)), 256,
               jax.ShapeDtypeStruct((_EH // 2, 128), jnp.float32),
               pl.BlockSpec((_R, 128), lambda i: (i, 0)), hh)
        for hh in range(NHALF)]
_TCM = [_mk_tc(_tc_mid_body,
               pl.BlockSpec((4, _R, 128), lambda i: (0, i, 0)), 640,
               jax.ShapeDtypeStruct((_EH // 2, 128), jnp.float32),
               pl.BlockSpec((_R, 128), lambda i: (i, 0)), hh)
        for hh in range(NHALF)]
_TCL = [_mk_tc(_tc_last_body,
               pl.BlockSpec((4, _R, 128), lambda i: (0, i, 0)), 640,
               jax.ShapeDtypeStruct((_EH // 2, 2), jnp.float32),
               pl.BlockSpec((_R, 2), lambda i: (i, 0)), hh,
               extra_specs=(pl.BlockSpec((128, H), lambda i: (0, 0)),
                            pl.BlockSpec((H, 8), lambda i: (0, 0)),
                            pl.BlockSpec((8, H), lambda i: (0, 0))))
        for hh in range(NHALF)]


def _lr(Wk, side):
    z = jnp.zeros_like(Wk)
    return jnp.concatenate([Wk, z] if side == 0 else [z, Wk], axis=1)


def _wbig_mid(W):
    blocks = []
    for k in range(5):
        Wk = W[k * 64:(k + 1) * 64]
        blocks += [_lr(Wk, 0), _lr(Wk, 1)]
    return jnp.concatenate(blocks, axis=0)


def _wbig0(W0):
    wx = jnp.zeros((64, H), jnp.float32).at[:CIN].set(W0[:CIN])
    combo = [jnp.zeros((16, H), jnp.float32).at[:CIN].set(
        W0[k * CIN:(k + 1) * CIN]) for k in range(1, 5)]
    blocks = [_lr(wx, 0), _lr(wx, 1)]
    blocks += [_lr(c, 0) for c in combo]
    blocks += [_lr(c, 1) for c in combo]
    return jnp.concatenate(blocks, axis=0)


def kernel(x, neighbors, W0, b0, g0, be0, W1, b1, g1, be1,
           W2, b2, g2, be2, W3, b3, g3, be3, cW1, cb1, cW2, cb2):
    a = jnp.pad(x, ((0, EP - E), (0, 0))).reshape(NCH, 2, 64, CIN)
    z53 = jnp.zeros((NCH, 64, 53), jnp.float32)
    x64 = jnp.concatenate([a[:, 0], z53, a[:, 1], z53],
                          axis=-1).reshape(EP // 2, 128)
    xr8 = x.reshape(E // 8, 88)
    z5 = jnp.zeros((E // 8, 5), jnp.float32)
    p16 = []
    for s in range(8):
        p16 += [xr8[:, s * 11:(s + 1) * 11], z5]
    x16 = jnp.pad(jnp.concatenate(p16, axis=1),
                  ((0, (EP - E) // 8), (0, 0))).reshape(EP, 16)
    nbq = (jnp.pad(neighbors, ((0, EP - E), (0, 0))).T
           .reshape(4 * NCH, B))

    def pack(b, g, be):
        return jnp.concatenate(
            [b[None], g[None], be[None], jnp.zeros((5, H), jnp.float32)], 0)

    cw1b = jnp.zeros((128, 64), jnp.float32)
    cw1b = cw1b.at[:64, :32].set(cW1).at[64:, 32:].set(cW1)
    w2b = jnp.zeros((H, 8), jnp.float32)
    w2b = w2b.at[:32, 0].set(cW2[:, 0]).at[32:, 1].set(cW2[:, 0])
    hp = jnp.zeros((8, H), jnp.float32)
    hp = hp.at[0, :32].set(cb1).at[0, 32:].set(cb1)
    hp = hp.at[2, 0].set(cb2[0])

    g16 = [_make_gather(16, hh) for hh in range(NHALF)]
    g64 = [_make_gather(64, hh) for hh in range(NHALF)]

    gx = [g16[hh](nbq, x16) for hh in range(NHALF)]
    w0v, p0v = _wbig0(W0), pack(b0, g0, be0)
    h = jnp.concatenate(
        [_TC0[hh](x64, gx[hh], w0v, p0v) for hh in range(NHALF)], axis=0)
    for (W, b, g, be) in ((W1, b1, g1, be1), (W2, b2, g2, be2)):
        tbl = h.reshape(EP, 64)
        gp = [g64[hh](nbq, tbl) for hh in range(NHALF)]
        Wb, pv = _wbig_mid(W), pack(b, g, be)
        h = jnp.concatenate(
            [_TCM[hh](h, gp[hh], Wb, pv) for hh in range(NHALF)], axis=0)
    tbl = h.reshape(EP, 64)
    gp = [g64[hh](nbq, tbl) for hh in range(NHALF)]
    Wb3, p3v = _wbig_mid(W3), pack(b3, g3, be3)
    out = jnp.concatenate(
        [_TCL[hh](h, gp[hh], Wb3, p3v, cw1b, w2b, hp)
         for hh in range(NHALF)], axis=0)
    return out.reshape(NCH, 64, 2).transpose(0, 2, 1).reshape(EP)[:E]


# speedup vs baseline: 1.0698x; 1.0481x over previous
import functools

import jax
import jax.numpy as jnp
from jax import lax
from jax.experimental import pallas as pl
from jax.experimental.pallas import tpu as pltpu
from jax.experimental.pallas import tpu_sc as plsc

E = 800000
CIN = 11
H = 64
B = 128
NC = 2
NS = 16
NW = NC * NS
CPW = 200
NHALF = 2
CPWH = CPW // NHALF
EP = NW * CPW * B
NCH = EP // B
BT = 2048
SB = 10
NBUF = 3


@functools.lru_cache(maxsize=None)
def _make_gather(C, half):
    off = half * (NCH // NHALF)
    epl = EP // NHALF
    out_shape = ((4, epl // 2, 128) if C == 64 else (epl // 2, 128))
    mesh = plsc.VectorSubcoreMesh(core_axis_name="c", subcore_axis_name="s",
                                  num_cores=NC)

    @functools.partial(
        pl.kernel,
        out_type=jax.ShapeDtypeStruct(out_shape, jnp.float32),
        mesh=mesh,
        compiler_params=pltpu.CompilerParams(use_tc_tiling_on_sc=False),
        scratch_types=[
            pltpu.VMEM((4, SB, B), jnp.int32),
            pltpu.VMEM((NBUF, 8, 64), jnp.int32),
            pltpu.VMEM((NBUF, 8, B // 2, C), jnp.float32),
        ] + [pltpu.SemaphoreType.DMA] * (2 * NBUF),
    )
    def gather_k(nb_hbm, h_hbm, out_hbm, raw_v, stage_v, gath_v, *sems):
        gsems, wsems = sems[:NBUF], sems[NBUF:]
        wid = lax.axis_index("c") * NS + lax.axis_index("s")
        base = off + wid * CPWH

        def stage_idx(k, s):
            for par in range(2):
                for j in range(4):
                    for pb in range(4):
                        e = raw_v[j, k, pl.ds(par * 64 + pb * 16, 16)]
                        r = (((e >> 7) << 7) + 2 * (e & 63) + ((e >> 6) & 1))
                        stage_v[s, par * 4 + j, pl.ds(pb * 16, 16)] = r

        def super_body(t, carry):
            ch0 = base + t * SB
            for j in range(4):
                pltpu.sync_copy(nb_hbm.at[pl.ds(j * NCH + ch0, SB)],
                                raw_v.at[j])
            gcp, wcp = {}, {}

            def dst(kk, j, par):
                row0 = (ch0 - off + kk) * (B // 2)
                if C == 64:
                    return out_hbm.at[j, pl.ds(row0, B // 2),
                                      pl.ds(par * 64, 64)]
                return out_hbm.at[pl.ds(row0, B // 2),
                                  pl.ds(par * 64 + j * 16, 16)]

            def start_stores(kk):
                ss = kk % NBUF
                for c in gcp[kk]:
                    c.wait()
                wcp[kk] = [
                    pltpu.async_copy(gath_v.at[ss, par * 4 + j],
                                     dst(kk, j, par), wsems[ss])
                    for par in range(2) for j in range(4)
                ]

            for k in range(SB):
                s = k % NBUF
                if k >= NBUF:
                    for c in wcp[k - NBUF]:
                        c.wait()
                if C == 64:
                    stage_idx(k, s)
                    idx = lambda j, par: stage_v.at[s, par * 4 + j]
                else:
                    idx = lambda j, par: raw_v.at[j, k, pl.ds(par * 64, 64)]
                gcp[k] = [
                    pltpu.async_copy(h_hbm.at[idx(j, par)],
                                     gath_v.at[s, par * 4 + j], gsems[s])
                    for par in range(2) for j in range(4)
                ]
                if k >= 2:
                    start_stores(k - 2)
            start_stores(SB - 2)
            start_stores(SB - 1)
            for k in range(SB - NBUF, SB):
                for c in wcp[k]:
                    c.wait()
            return carry

        lax.fori_loop(0, CPWH // SB, super_body, None)

    return gather_k


def _ln_relu_pair(z, p_ref):
    halves = []
    for h0 in (0, 64):
        zz = z[:, h0:h0 + 64] + p_ref[0][None, :]
        m = jnp.mean(zz, axis=1, keepdims=True)
        zc = zz - m
        v = jnp.mean(zc * zc, axis=1, keepdims=True)
        halves.append(jnp.maximum(
            zc * lax.rsqrt(v + 1e-5) * p_ref[1][None, :]
            + p_ref[2][None, :], 0.0))
    return jnp.concatenate(halves, axis=1)


def _mid_act(h_ref, g_ref, w_ref, p_ref):
    hb = h_ref[...]
    g0, g1, g2, g3 = g_ref[0], g_ref[1], g_ref[2], g_ref[3]
    feats = (hb, g0 + g2, jnp.abs(g0 - g2), g1 + g3, jnp.abs(g1 - g3))
    z = None
    for k, f in enumerate(feats):
        zk = jnp.dot(f, w_ref[k * 128:(k + 1) * 128],
                     preferred_element_type=jnp.float32)
        z = zk if z is None else z + zk
    return _ln_relu_pair(z, p_ref) + hb


def _tc0_body(x_ref, g_ref, w_ref, p_ref, o_ref):
    xb = x_ref[...]
    gx = g_ref[...]
    pieces = [xb]
    for h0 in (0, 64):
        n0 = gx[:, h0:h0 + 16]
        n1 = gx[:, h0 + 16:h0 + 32]
        n2 = gx[:, h0 + 32:h0 + 48]
        n3 = gx[:, h0 + 48:h0 + 64]
        pieces += [n0 + n2, jnp.abs(n0 - n2), n1 + n3, jnp.abs(n1 - n3)]
    f = jnp.concatenate(pieces, axis=1)
    z = jnp.dot(f, w_ref[...], preferred_element_type=jnp.float32)
    o_ref[...] = _ln_relu_pair(z, p_ref)


def _tc_mid_body(h_ref, g_ref, w_ref, p_ref, o_ref):
    o_ref[...] = _mid_act(h_ref, g_ref, w_ref, p_ref)


def _tc_last_body(h_ref, g_ref, w_ref, p_ref, cw1_ref, w2_ref, hp_ref, o_ref):
    a = _mid_act(h_ref, g_ref, w_ref, p_ref)
    t = jnp.maximum(
        jnp.dot(a, cw1_ref[...], preferred_element_type=jnp.float32)
        + hp_ref[0][None, :], 0.0)
    o2 = jnp.dot(t, w2_ref[...], preferred_element_type=jnp.float32)
    o_ref[...] = o2[:, :2] + hp_ref[2, 0]


_R = BT // 2
_GH = EP // BT // NHALF


def _mk_tc(body, g_spec, kw, out_shape, out_spec, half, extra_specs=()):
    hoff = half * _GH
    return pl.pallas_call(
        body,
        grid=(_GH,),
        in_specs=[
            pl.BlockSpec((_R, 128), lambda i: (i + hoff, 0)),
            g_spec,
            pl.BlockSpec((kw, 128), lambda i: (0, 0)),
            pl.BlockSpec((8, H), lambda i: (0, 0)),
            *extra_specs,
        ],
        out_specs=out_spec,
        out_shape=out_shape,
    )


_EH = EP // NHALF
_TC0 = [_mk_tc(_tc0_body,
               pl.BlockSpec((_R, 128), lambda i: (i, 0)), 256,
               jax.ShapeDtypeStruct((_EH // 2, 128), jnp.float32),
               pl.BlockSpec((_R, 128), lambda i: (i, 0)), hh)
        for hh in range(NHALF)]
_TCM = [_mk_tc(_tc_mid_body,
               pl.BlockSpec((4, _R, 128), lambda i: (0, i, 0)), 640,
               jax.ShapeDtypeStruct((_EH // 2, 128), jnp.float32),
               pl.BlockSpec((_R, 128), lambda i: (i, 0)), hh)
        for hh in range(NHALF)]
_TCL = [_mk_tc(_tc_last_body,
               pl.BlockSpec((4, _R, 128), lambda i: (0, i, 0)), 640,
               jax.ShapeDtypeStruct((_EH // 2, 2), jnp.float32),
               pl.BlockSpec((_R, 2), lambda i: (i, 0)), hh,
               extra_specs=(pl.BlockSpec((128, H), lambda i: (0, 0)),
                            pl.BlockSpec((H, 8), lambda i: (0, 0)),
                            pl.BlockSpec((8, H), lambda i: (0, 0))))
        for hh in range(NHALF)]


def _lr(Wk, side):
    z = jnp.zeros_like(Wk)
    return jnp.concatenate([Wk, z] if side == 0 else [z, Wk], axis=1)


def _wbig_mid(W):
    blocks = []
    for k in range(5):
        Wk = W[k * 64:(k + 1) * 64]
        blocks += [_lr(Wk, 0), _lr(Wk, 1)]
    return jnp.concatenate(blocks, axis=0)


def _wbig0(W0):
    wx = jnp.zeros((64, H), jnp.float32).at[:CIN].set(W0[:CIN])
    combo = [jnp.zeros((16, H), jnp.float32).at[:CIN].set(
        W0[k * CIN:(k + 1) * CIN]) for k in range(1, 5)]
    blocks = [_lr(wx, 0), _lr(wx, 1)]
    blocks += [_lr(c, 0) for c in combo]
    blocks += [_lr(c, 1) for c in combo]
    return jnp.concatenate(blocks, axis=0)


def kernel(x, neighbors, W0, b0, g0, be0, W1, b1, g1, be1,
           W2, b2, g2, be2, W3, b3, g3, be3, cW1, cb1, cW2, cb2):
    a = jnp.pad(x, ((0, EP - E), (0, 0))).reshape(NCH, 2, 64, CIN)
    z53 = jnp.zeros((NCH, 64, 53), jnp.float32)
    x64 = jnp.concatenate([a[:, 0], z53, a[:, 1], z53],
                          axis=-1).reshape(EP // 2, 128)
    xr8 = x.reshape(E // 8, 88)
    z5 = jnp.zeros((E // 8, 5), jnp.float32)
    p16 = []
    for s in range(8):
        p16 += [xr8[:, s * 11:(s + 1) * 11], z5]
    x16 = jnp.pad(jnp.concatenate(p16, axis=1),
                  ((0, (EP - E) // 8), (0, 0))).reshape(EP, 16)
    nbq = (jnp.pad(neighbors, ((0, EP - E), (0, 0))).T
           .reshape(4 * NCH, B))

    def pack(b, g, be):
        return jnp.concatenate(
            [b[None], g[None], be[None], jnp.zeros((5, H), jnp.float32)], 0)

    cw1b = jnp.zeros((128, 64), jnp.float32)
    cw1b = cw1b.at[:64, :32].set(cW1).at[64:, 32:].set(cW1)
    w2b = jnp.zeros((H, 8), jnp.float32)
    w2b = w2b.at[:32, 0].set(cW2[:, 0]).at[32:, 1].set(cW2[:, 0])
    hp = jnp.zeros((8, H), jnp.float32)
    hp = hp.at[0, :32].set(cb1).at[0, 32:].set(cb1)
    hp = hp.at[2, 0].set(cb2[0])

    g16 = [_make_gather(16, hh) for hh in range(NHALF)]
    g64 = [_make_gather(64, hh) for hh in range(NHALF)]

    gx = [g16[hh](nbq, x16) for hh in range(NHALF)]
    w0v, p0v = _wbig0(W0), pack(b0, g0, be0)
    h = jnp.concatenate(
        [_TC0[hh](x64, gx[hh], w0v, p0v) for hh in range(NHALF)], axis=0)
    for (W, b, g, be) in ((W1, b1, g1, be1), (W2, b2, g2, be2)):
        tbl = h.reshape(EP, 64)
        gp = [g64[hh](nbq, tbl) for hh in range(NHALF)]
        Wb, pv = _wbig_mid(W), pack(b, g, be)
        h = jnp.concatenate(
            [_TCM[hh](h, gp[hh], Wb, pv) for hh in range(NHALF)], axis=0)
    tbl = h.reshape(EP, 64)
    gp = [g64[hh](nbq, tbl) for hh in range(NHALF)]
    Wb3, p3v = _wbig_mid(W3), pack(b3, g3, be3)
    out = jnp.concatenate(
        [_TCL[hh](h, gp[hh], Wb3, p3v, cw1b, w2b, hp)
         for hh in range(NHALF)], axis=0)
    return out.reshape(NCH, 64, 2).transpose(0, 2, 1).reshape(EP)[:E]


# speedup vs baseline: 1.0792x; 1.0089x over previous
import functools

import jax
import jax.numpy as jnp
from jax import lax
from jax.experimental import pallas as pl
from jax.experimental.pallas import tpu as pltpu
from jax.experimental.pallas import tpu_sc as plsc

E = 800000
CIN = 11
H = 64
B = 128
NC = 2
NS = 16
NW = NC * NS
CPW = 200
NHALF = 2
CPWH = CPW // NHALF
EP = NW * CPW * B
NCH = EP // B
BT = 4096
SB = 10
NBUF = 3


@functools.lru_cache(maxsize=None)
def _make_gather(C, half):
    off = half * (NCH // NHALF)
    epl = EP // NHALF
    out_shape = ((4, epl // 2, 128) if C == 64 else (epl // 2, 128))
    mesh = plsc.VectorSubcoreMesh(core_axis_name="c", subcore_axis_name="s",
                                  num_cores=NC)

    @functools.partial(
        pl.kernel,
        out_type=jax.ShapeDtypeStruct(out_shape, jnp.float32),
        mesh=mesh,
        compiler_params=pltpu.CompilerParams(use_tc_tiling_on_sc=False),
        scratch_types=[
            pltpu.VMEM((4, SB, B), jnp.int32),
            pltpu.VMEM((NBUF, 8, 64), jnp.int32),
            pltpu.VMEM((NBUF, 8, B // 2, C), jnp.float32),
        ] + [pltpu.SemaphoreType.DMA] * (2 * NBUF),
    )
    def gather_k(nb_hbm, h_hbm, out_hbm, raw_v, stage_v, gath_v, *sems):
        gsems, wsems = sems[:NBUF], sems[NBUF:]
        wid = lax.axis_index("c") * NS + lax.axis_index("s")
        base = off + wid * CPWH

        def stage_idx(k, s):
            for par in range(2):
                for j in range(4):
                    for pb in range(4):
                        e = raw_v[j, k, pl.ds(par * 64 + pb * 16, 16)]
                        r = (((e >> 7) << 7) + 2 * (e & 63) + ((e >> 6) & 1))
                        stage_v[s, par * 4 + j, pl.ds(pb * 16, 16)] = r

        def super_body(t, carry):
            ch0 = base + t * SB
            for j in range(4):
                pltpu.sync_copy(nb_hbm.at[pl.ds(j * NCH + ch0, SB)],
                                raw_v.at[j])
            gcp, wcp = {}, {}

            def dst(kk, j, par):
                row0 = (ch0 - off + kk) * (B // 2)
                if C == 64:
                    return out_hbm.at[j, pl.ds(row0, B // 2),
                                      pl.ds(par * 64, 64)]
                return out_hbm.at[pl.ds(row0, B // 2),
                                  pl.ds(par * 64 + j * 16, 16)]

            def start_stores(kk):
                ss = kk % NBUF
                for c in gcp[kk]:
                    c.wait()
                wcp[kk] = [
                    pltpu.async_copy(gath_v.at[ss, par * 4 + j],
                                     dst(kk, j, par), wsems[ss])
                    for par in range(2) for j in range(4)
                ]

            for k in range(SB):
                s = k % NBUF
                if k >= NBUF:
                    for c in wcp[k - NBUF]:
                        c.wait()
                if C == 64:
                    stage_idx(k, s)
                    idx = lambda j, par: stage_v.at[s, par * 4 + j]
                else:
                    idx = lambda j, par: raw_v.at[j, k, pl.ds(par * 64, 64)]
                gcp[k] = [
                    pltpu.async_copy(h_hbm.at[idx(j, par)],
                                     gath_v.at[s, par * 4 + j], gsems[s])
                    for par in range(2) for j in range(4)
                ]
                if k >= 2:
                    start_stores(k - 2)
            start_stores(SB - 2)
            start_stores(SB - 1)
            for k in range(SB - NBUF, SB):
                for c in wcp[k]:
                    c.wait()
            return carry

        lax.fori_loop(0, CPWH // SB, super_body, None)

    return gather_k


def _ln_relu_pair(z, p_ref):
    halves = []
    for h0 in (0, 64):
        zz = z[:, h0:h0 + 64] + p_ref[0][None, :]
        m = jnp.mean(zz, axis=1, keepdims=True)
        zc = zz - m
        v = jnp.mean(zc * zc, axis=1, keepdims=True)
        halves.append(jnp.maximum(
            zc * lax.rsqrt(v + 1e-5) * p_ref[1][None, :]
            + p_ref[2][None, :], 0.0))
    return jnp.concatenate(halves, axis=1)


def _mid_act(h_ref, g_ref, w_ref, p_ref):
    hb = h_ref[...]
    g0, g1, g2, g3 = g_ref[0], g_ref[1], g_ref[2], g_ref[3]
    feats = (hb, g0 + g2, jnp.abs(g0 - g2), g1 + g3, jnp.abs(g1 - g3))
    z = None
    for k, f in enumerate(feats):
        zk = jnp.dot(f, w_ref[k * 128:(k + 1) * 128],
                     preferred_element_type=jnp.float32)
        z = zk if z is None else z + zk
    return _ln_relu_pair(z, p_ref) + hb


def _tc0_body(x_ref, g_ref, w_ref, p_ref, o_ref):
    xb = x_ref[...]
    gx = g_ref[...]
    pieces = [xb]
    for h0 in (0, 64):
        n0 = gx[:, h0:h0 + 16]
        n1 = gx[:, h0 + 16:h0 + 32]
        n2 = gx[:, h0 + 32:h0 + 48]
        n3 = gx[:, h0 + 48:h0 + 64]
        pieces += [n0 + n2, jnp.abs(n0 - n2), n1 + n3, jnp.abs(n1 - n3)]
    f = jnp.concatenate(pieces, axis=1)
    z = jnp.dot(f, w_ref[...], preferred_element_type=jnp.float32)
    o_ref[...] = _ln_relu_pair(z, p_ref)


def _tc_mid_body(h_ref, g_ref, w_ref, p_ref, o_ref):
    o_ref[...] = _mid_act(h_ref, g_ref, w_ref, p_ref)


def _tc_last_body(h_ref, g_ref, w_ref, p_ref, cw1_ref, w2_ref, hp_ref, o_ref):
    a = _mid_act(h_ref, g_ref, w_ref, p_ref)
    t = jnp.maximum(
        jnp.dot(a, cw1_ref[...], preferred_element_type=jnp.float32)
        + hp_ref[0][None, :], 0.0)
    o2 = jnp.dot(t, w2_ref[...], preferred_element_type=jnp.float32)
    o_ref[...] = o2[:, :2] + hp_ref[2, 0]


_R = BT // 2
_GH = EP // BT // NHALF


def _mk_tc(body, g_spec, kw, out_shape, out_spec, half, extra_specs=()):
    hoff = half * _GH
    return pl.pallas_call(
        body,
        grid=(_GH,),
        in_specs=[
            pl.BlockSpec((_R, 128), lambda i: (i + hoff, 0)),
            g_spec,
            pl.BlockSpec((kw, 128), lambda i: (0, 0)),
            pl.BlockSpec((8, H), lambda i: (0, 0)),
            *extra_specs,
        ],
        out_specs=out_spec,
        out_shape=out_shape,
    )


_EH = EP // NHALF
_TC0 = [_mk_tc(_tc0_body,
               pl.BlockSpec((_R, 128), lambda i: (i, 0)), 256,
               jax.ShapeDtypeStruct((_EH // 2, 128), jnp.float32),
               pl.BlockSpec((_R, 128), lambda i: (i, 0)), hh)
        for hh in range(NHALF)]
_TCM = [_mk_tc(_tc_mid_body,
               pl.BlockSpec((4, _R, 128), lambda i: (0, i, 0)), 640,
               jax.ShapeDtypeStruct((_EH // 2, 128), jnp.float32),
               pl.BlockSpec((_R, 128), lambda i: (i, 0)), hh)
        for hh in range(NHALF)]
_TCL = [_mk_tc(_tc_last_body,
               pl.BlockSpec((4, _R, 128), lambda i: (0, i, 0)), 640,
               jax.ShapeDtypeStruct((_EH // 2, 2), jnp.float32),
               pl.BlockSpec((_R, 2), lambda i: (i, 0)), hh,
               extra_specs=(pl.BlockSpec((128, H), lambda i: (0, 0)),
                            pl.BlockSpec((H, 8), lambda i: (0, 0)),
                            pl.BlockSpec((8, H), lambda i: (0, 0))))
        for hh in range(NHALF)]


def _lr(Wk, side):
    z = jnp.zeros_like(Wk)
    return jnp.concatenate([Wk, z] if side == 0 else [z, Wk], axis=1)


def _wbig_mid(W):
    blocks = []
    for k in range(5):
        Wk = W[k * 64:(k + 1) * 64]
        blocks += [_lr(Wk, 0), _lr(Wk, 1)]
    return jnp.concatenate(blocks, axis=0)


def _wbig0(W0):
    wx = jnp.zeros((64, H), jnp.float32).at[:CIN].set(W0[:CIN])
    combo = [jnp.zeros((16, H), jnp.float32).at[:CIN].set(
        W0[k * CIN:(k + 1) * CIN]) for k in range(1, 5)]
    blocks = [_lr(wx, 0), _lr(wx, 1)]
    blocks += [_lr(c, 0) for c in combo]
    blocks += [_lr(c, 1) for c in combo]
    return jnp.concatenate(blocks, axis=0)


def kernel(x, neighbors, W0, b0, g0, be0, W1, b1, g1, be1,
           W2, b2, g2, be2, W3, b3, g3, be3, cW1, cb1, cW2, cb2):
    a = jnp.pad(x, ((0, EP - E), (0, 0))).reshape(NCH, 2, 64, CIN)
    z53 = jnp.zeros((NCH, 64, 53), jnp.float32)
    x64 = jnp.concatenate([a[:, 0], z53, a[:, 1], z53],
                          axis=-1).reshape(EP // 2, 128)
    xr8 = x.reshape(E // 8, 88)
    z5 = jnp.zeros((E // 8, 5), jnp.float32)
    p16 = []
    for s in range(8):
        p16 += [xr8[:, s * 11:(s + 1) * 11], z5]
    x16 = jnp.pad(jnp.concatenate(p16, axis=1),
                  ((0, (EP - E) // 8), (0, 0))).reshape(EP, 16)
    nbq = (jnp.pad(neighbors, ((0, EP - E), (0, 0))).T
           .reshape(4 * NCH, B))

    def pack(b, g, be):
        return jnp.concatenate(
            [b[None], g[None], be[None], jnp.zeros((5, H), jnp.float32)], 0)

    cw1b = jnp.zeros((128, 64), jnp.float32)
    cw1b = cw1b.at[:64, :32].set(cW1).at[64:, 32:].set(cW1)
    w2b = jnp.zeros((H, 8), jnp.float32)
    w2b = w2b.at[:32, 0].set(cW2[:, 0]).at[32:, 1].set(cW2[:, 0])
    hp = jnp.zeros((8, H), jnp.float32)
    hp = hp.at[0, :32].set(cb1).at[0, 32:].set(cb1)
    hp = hp.at[2, 0].set(cb2[0])

    g16 = [_make_gather(16, hh) for hh in range(NHALF)]
    g64 = [_make_gather(64, hh) for hh in range(NHALF)]

    gx = [g16[hh](nbq, x16) for hh in range(NHALF)]
    w0v, p0v = _wbig0(W0), pack(b0, g0, be0)
    h = jnp.concatenate(
        [_TC0[hh](x64, gx[hh], w0v, p0v) for hh in range(NHALF)], axis=0)
    for (W, b, g, be) in ((W1, b1, g1, be1), (W2, b2, g2, be2)):
        tbl = h.reshape(EP, 64)
        gp = [g64[hh](nbq, tbl) for hh in range(NHALF)]
        Wb, pv = _wbig_mid(W), pack(b, g, be)
        h = jnp.concatenate(
            [_TCM[hh](h, gp[hh], Wb, pv) for hh in range(NHALF)], axis=0)
    tbl = h.reshape(EP, 64)
    gp = [g64[hh](nbq, tbl) for hh in range(NHALF)]
    Wb3, p3v = _wbig_mid(W3), pack(b3, g3, be3)
    out = jnp.concatenate(
        [_TCL[hh](h, gp[hh], Wb3, p3v, cw1b, w2b, hp)
         for hh in range(NHALF)], axis=0)
    return out.reshape(NCH, 64, 2).transpose(0, 2, 1).reshape(EP)[:E]
